# baseline (speedup 1.0000x reference)
import jax
import jax.numpy as jnp
from jax import lax
from jax.experimental import pallas as pl
from jax.experimental.pallas import tpu as pltpu
from jax.experimental.pallas import tpu_sc as plsc

H = 128
N = 10000
E = 320000

NC = 2
NS = 16
NW = NC * NS
EPW = E // NW
C = 40
NCHUNK = EPW // C
RING = 4
IRING = 8
NMAIN = (NCHUNK // IRING) * IRING
NPAD = 10240
RPT = NPAD // NS
JPT = RPT // C


def _relu(x):
    return jnp.maximum(x, 0.0)


def _mm(a, b):
    return jax.lax.dot_general(
        a, b, (((1,), (0,)), ((), ())), preferred_element_type=jnp.float32
    )


def q_body(ea, wew, web, b1a, b1b, b2a, b2b, b3a, b3b, q1, q2, q3):
    he = _relu(
        ea[:, 0:1] * wew[0:1, :] + ea[:, 1:2] * wew[1:2, :] + web[0:1, :]
    )

    heb = he.astype(jnp.bfloat16)

    def packq(ba, bb, qo):
        qa = jax.lax.bitcast_convert_type(
            _mm(heb, ba[...]).astype(jnp.bfloat16), jnp.uint16).astype(jnp.int32)
        qb = jax.lax.bitcast_convert_type(
            _mm(heb, bb[...]).astype(jnp.bfloat16), jnp.uint16).astype(jnp.int32)
        qo[...] = jax.lax.shift_left(qb, 16) | qa

    packq(b1a, b1b, q1)
    packq(b2a, b2b, q2)
    packq(b3a, b3b, q3)


def pre_body(stx, sx, e1w, e1b, e2w, e2b, f1w, f1b, f2w, f2b,
             f2st, f2sbw, f2s_b, mw, mb, hst_o, hf_o, p_o):
    a = _relu(
        stx[:, 0:1] * e1w[0:1, :]
        + stx[:, 1:2] * e1w[1:2, :]
        + stx[:, 2:3] * e1w[2:3, :]
        + e1b[0:1, :]
    )
    h_st = _relu(_mm(a, e2w[...]) + e2b[0:1, :])
    b0 = _relu(sx[:, 0:1] * f1w[0:1, :] + f1b[0:1, :])
    h_f = _relu(_mm(b0, f2w[...]) + f2b[0:1, :])
    h_st = h_st + _relu(_mm(h_st, f2st[...]) + _mm(h_f, f2sbw[...]) + f2s_b[0:1, :])
    hst_o[...] = h_st
    hf_o[...] = h_f
    p_o[...] = _mm(h_st, mw[...]) + mb[0:1, :]


def mid_body(hst, hf, parts, updt, updbw, upd_b, s2ft, s2fbw, s2f_b,
             f2st, f2sbw, f2s_b, mw, mb, hst_o, hf_o, p_o):
    agg = parts[0:N, :] + parts[NPAD:NPAD + N, :]
    h_st = hst[...]
    h_st = h_st + _relu(_mm(h_st, updt[...]) + _mm(agg, updbw[...]) + upd_b[0:1, :])
    h_f = hf[...]
    h_f = h_f + _relu(_mm(h_f, s2ft[...]) + _mm(h_st, s2fbw[...]) + s2f_b[0:1, :])
    h_st2 = h_st + _relu(_mm(h_st, f2st[...]) + _mm(h_f, f2sbw[...]) + f2s_b[0:1, :])
    hst_o[...] = h_st2
    hf_o[...] = h_f
    p_o[...] = _mm(h_st2, mw[...]) + mb[0:1, :]


def fin_body(hst, hf, parts, updt, updbw, upd_b, s2ft, s2fbw, s2f_b,
             h1t, h1bw, h1_b, h2w, h2b, tot_o, local_o):
    agg = parts[0:N, :] + parts[NPAD:NPAD + N, :]
    h_st = hst[...]
    h_st = h_st + _relu(_mm(h_st, updt[...]) + _mm(agg, updbw[...]) + upd_b[0:1, :])
    h_f = hf[...]
    h_f = h_f + _relu(_mm(h_f, s2ft[...]) + _mm(h_st, s2fbw[...]) + s2f_b[0:1, :])
    z = _relu(_mm(h_st, h1t[...]) + _mm(h_f, h1bw[...]) + h1_b[0:1, :])
    local = _mm(z, h2w[...]) + h2b[0:1, :]
    local_o[...] = local
    tot_o[...] = jnp.sum(local).reshape(1, 1)


def sc_body(p_hbm, q_hbm, src_hbm, dst_hbm, out_hbm,
            isrc, idst, rows, qbuf, acc, gsems, qsems, ssems, isems, dsems):
    c = lax.axis_index("c")
    s = lax.axis_index("s")
    wid = s * NC + c

    def ds8(start, n):
        return pl.ds(pl.multiple_of(start, 8), n)

    zero16 = jnp.zeros((16,), jnp.float32)

    def zrow(i, carry):
        for k in range(H // 16):
            rows[0, i, pl.ds(k * 16, 16)] = zero16
        return carry

    lax.fori_loop(0, C, zrow, 0)
    for j in range(JPT):
        pltpu.sync_copy(rows.at[0], acc.at[pl.ds(s * RPT + j * C, C)])
    plsc.subcore_barrier()

    def issue_idx(g, bi):
        base = ds8(wid * EPW + g * C, C)
        pltpu.async_copy(src_hbm.at[base], isrc.at[bi], isems.at[bi])
        pltpu.async_copy(dst_hbm.at[base], idst.at[bi], dsems.at[bi])

    def wait_idx(g, bi):
        base = ds8(wid * EPW + g * C, C)
        pltpu.make_async_copy(src_hbm.at[base], isrc.at[bi],
                              isems.at[bi]).wait()
        pltpu.make_async_copy(dst_hbm.at[base], idst.at[bi],
                              dsems.at[bi]).wait()

    def issue_gq(g, b, bi):
        pltpu.async_copy(p_hbm.at[isrc.at[bi]], rows.at[b], gsems.at[b])
        pltpu.async_copy(q_hbm.at[ds8(wid * EPW + g * C, C)],
                         qbuf.at[b], qsems.at[b])

    def wait_gq(g, b, bi):
        pltpu.make_async_copy(p_hbm.at[isrc.at[bi]], rows.at[b],
                              gsems.at[b]).wait()
        pltpu.make_async_copy(q_hbm.at[ds8(wid * EPW + g * C, C)],
                              qbuf.at[b], qsems.at[b]).wait()

    def wait_scatter(b, bi):
        pltpu.make_async_copy(rows.at[b], acc.at[idst.at[bi]],
                              ssems.at[b]).wait()

    mask_hi = jnp.full((16,), -65536, jnp.int32)
    sh16 = jnp.full((16,), 16, jnp.int32)

    def compute_scatter(b, bi):
        def erow(e, carry):
            for k in range(H // 32):
                qi = qbuf[b, e, pl.ds(k * 16, 16)]
                lo = jax.lax.bitcast_convert_type(
                    jax.lax.shift_left(qi, sh16), jnp.float32)
                hi = jax.lax.bitcast_convert_type(
                    jax.lax.bitwise_and(qi, mask_hi), jnp.float32)
                sl = pl.ds(k * 32, 16)
                sh = pl.ds(k * 32 + 16, 16)
                rows[b, e, sl] = jnp.maximum(rows[b, e, sl] + lo, 0.0)
                rows[b, e, sh] = jnp.maximum(rows[b, e, sh] + hi, 0.0)
            return carry

        lax.fori_loop(0, C, erow, 0)
        pltpu.async_copy(rows.at[b], acc.at[idst.at[bi]], ssems.at[b],
                         add=True)

    for x in range(4):
        issue_idx(x, x)
    for x in range(2):
        wait_idx(x, x)
        issue_gq(x, x, x)

    def outer(gg, carry):
        for j in range(IRING):
            g = gg * IRING + j
            b = j % RING

            @pl.when(g + 4 < NCHUNK)
            def _():
                issue_idx(g + 4, (j + 4) % IRING)

            @pl.when(g >= 2)
            def _():
                wait_scatter((j + 2) % RING, (j + 6) % IRING)

            wait_idx(g + 2, (j + 2) % IRING)
            issue_gq(g + 2, (j + 2) % RING, (j + 2) % IRING)

            wait_gq(g, b, j)
            compute_scatter(b, j)
        return carry

    lax.fori_loop(0, NMAIN // IRING, outer, 0)

    for g in range(NMAIN, NCHUNK):
        j = g % IRING
        wait_scatter((j + 2) % RING, (j + 6) % IRING)
        wait_gq(g, j % RING, j)
        compute_scatter(j % RING, j)
    for g in range(NCHUNK - 2, NCHUNK):
        wait_scatter(g % RING, g % IRING)

    plsc.subcore_barrier()
    for j in range(JPT):
        r0 = s * RPT + j * C
        pltpu.sync_copy(acc.at[pl.ds(r0, C)], out_hbm.at[pl.ds(c * NPAD + r0, C)])


def _sc_edge(p, q, src1, dst1):
    return pl.kernel(
        sc_body,
        out_type=jax.ShapeDtypeStruct((2 * NPAD, H), jnp.float32),
        mesh=plsc.VectorSubcoreMesh(core_axis_name="c", subcore_axis_name="s"),
        scratch_types=[
            pltpu.VMEM((IRING, C), jnp.int32),
            pltpu.VMEM((IRING, C), jnp.int32),
            pltpu.VMEM((RING, C, H), jnp.float32),
            pltpu.VMEM((RING, C, H // 2), jnp.int32),
            pltpu.VMEM_SHARED((NPAD, H), jnp.float32),
            pltpu.SemaphoreType.DMA((RING,)),
            pltpu.SemaphoreType.DMA((RING,)),
            pltpu.SemaphoreType.DMA((RING,)),
            pltpu.SemaphoreType.DMA((IRING,)),
            pltpu.SemaphoreType.DMA((IRING,)),
        ],
    )(p, q, src1, dst1)


def _full(shape):
    return pl.BlockSpec(shape, lambda i: tuple(0 for _ in shape))


_TE = 4000


def _q_call(ea, wew, web, bs):
    out = jax.ShapeDtypeStruct((E, H // 2), jnp.int32)
    return pl.pallas_call(
        q_body,
        grid=(E // _TE,),
        in_specs=[
            pl.BlockSpec((_TE, 2), lambda i: (i, 0)),
            _full((2, H)), _full((1, H)),
        ] + [_full((H, H // 2))] * 6,
        out_specs=[pl.BlockSpec((_TE, H // 2), lambda i: (i, 0))] * 3,
        out_shape=[out, out, out],
    )(ea, wew, web, *bs)


def _node_call(body, outs, *args):
    return pl.pallas_call(body, out_shape=list(outs))(*args)


def kernel(st_x, scalar_x, edge_index, edge_attr, params):
    f32 = jnp.float32
    src1 = edge_index[0]
    dst1 = edge_index[1]

    (e1w, e1b), (e2w, e2b) = params["st_enc"]
    (f1w, f1b), (f2w, f2b) = params["f_enc"]
    wew, web = params["edge_enc"]
    blks = params["blocks"]
    (h1w, h1b), (h2w, h2b) = params["head"]

    def row(b):
        return b.reshape(1, -1)

    def split(w):
        return w[:H], w[H:]

    msg_t, msg_b, msg_bias = [], [], []
    f2s_t, f2s_b, f2s_bias = [], [], []
    upd_t, upd_b, upd_bias = [], [], []
    s2f_t, s2f_b, s2f_bias = [], [], []
    for bl in blks:
        t, b_ = split(bl["msg"][0]); msg_t.append(t); msg_b.append(b_)
        msg_bias.append(row(bl["msg"][1]))
        t, b_ = split(bl["f2s"][0]); f2s_t.append(t); f2s_b.append(b_)
        f2s_bias.append(row(bl["f2s"][1]))
        t, b_ = split(bl["upd"][0]); upd_t.append(t); upd_b.append(b_)
        upd_bias.append(row(bl["upd"][1]))
        t, b_ = split(bl["s2f"][0]); s2f_t.append(t); s2f_b.append(b_)
        s2f_bias.append(row(bl["s2f"][1]))
    h1t, h1bw = split(h1w)

    cols_a = [32 * k + j for k in range(H // 32) for j in range(16)]
    cols_b = [c + 16 for c in cols_a]
    bs = []
    for k in range(3):
        bs += [msg_b[k][:, cols_a].astype(jnp.bfloat16),
               msg_b[k][:, cols_b].astype(jnp.bfloat16)]
    q1, q2, q3 = _q_call(edge_attr, wew, row(web), bs)

    nh = jax.ShapeDtypeStruct((N, H), f32)
    hst, hf, p1 = _node_call(
        pre_body, (nh, nh, nh),
        st_x, scalar_x, e1w, row(e1b), e2w, row(e2b),
        f1w, row(f1b), f2w, row(f2b),
        f2s_t[0], f2s_b[0], f2s_bias[0], msg_t[0], msg_bias[0],
    )

    parts = _sc_edge(p1, q1, src1, dst1)
    hst, hf, p2 = _node_call(
        mid_body, (nh, nh, nh),
        hst, hf, parts,
        upd_t[0], upd_b[0], upd_bias[0],
        s2f_t[0], s2f_b[0], s2f_bias[0],
        f2s_t[1], f2s_b[1], f2s_bias[1], msg_t[1], msg_bias[1],
    )

    parts = _sc_edge(p2, q2, src1, dst1)
    hst, hf, p3 = _node_call(
        mid_body, (nh, nh, nh),
        hst, hf, parts,
        upd_t[1], upd_b[1], upd_bias[1],
        s2f_t[1], s2f_b[1], s2f_bias[1],
        f2s_t[2], f2s_b[2], f2s_bias[2], msg_t[2], msg_bias[2],
    )

    parts = _sc_edge(p3, q3, src1, dst1)
    tot, local = _node_call(
        fin_body,
        (jax.ShapeDtypeStruct((1, 1), f32), jax.ShapeDtypeStruct((N, 1), f32)),
        hst, hf, parts,
        upd_t[2], upd_b[2], upd_bias[2],
        s2f_t[2], s2f_b[2], s2f_bias[2],
        h1t, h1bw, row(h1b), h2w, row(h2b),
    )
    return tot.reshape(1), local


# speedup vs baseline: 1.0092x; 1.0092x over previous
import jax
import jax.numpy as jnp
from jax import lax
from jax.experimental import pallas as pl
from jax.experimental.pallas import tpu as pltpu
from jax.experimental.pallas import tpu_sc as plsc

H = 128
N = 10000
E = 320000

NC = 2
NS = 16
NW = NC * NS
EPW = E // NW
C = 40
NCHUNK = EPW // C
RING = 4
IRING = 8
NMAIN = (NCHUNK // IRING) * IRING
NPAD = 10240
RPT = NPAD // NS
JPT = RPT // C


def _relu(x):
    return jnp.maximum(x, 0.0)


def _mm(a, b):
    return jax.lax.dot_general(
        a, b, (((1,), (0,)), ((), ())), preferred_element_type=jnp.float32
    )


def q_body(ea, wew, web, b1a, b1b, b2a, b2b, b3a, b3b, q1, q2, q3):
    he = _relu(
        ea[:, 0:1] * wew[0:1, :] + ea[:, 1:2] * wew[1:2, :] + web[0:1, :]
    )

    def packq(ba, bb, qo):
        qa = jax.lax.bitcast_convert_type(
            _mm(he, ba[...]).astype(jnp.bfloat16), jnp.uint16).astype(jnp.int32)
        qb = jax.lax.bitcast_convert_type(
            _mm(he, bb[...]).astype(jnp.bfloat16), jnp.uint16).astype(jnp.int32)
        qo[...] = jax.lax.shift_left(qb, 16) | qa

    packq(b1a, b1b, q1)
    packq(b2a, b2b, q2)
    packq(b3a, b3b, q3)


def pre_body(stx, sx, e1w, e1b, e2w, e2b, f1w, f1b, f2w, f2b,
             f2st, f2sbw, f2s_b, mw, mb, hst_o, hf_o, p_o):
    a = _relu(
        stx[:, 0:1] * e1w[0:1, :]
        + stx[:, 1:2] * e1w[1:2, :]
        + stx[:, 2:3] * e1w[2:3, :]
        + e1b[0:1, :]
    )
    h_st = _relu(_mm(a, e2w[...]) + e2b[0:1, :])
    b0 = _relu(sx[:, 0:1] * f1w[0:1, :] + f1b[0:1, :])
    h_f = _relu(_mm(b0, f2w[...]) + f2b[0:1, :])
    h_st = h_st + _relu(_mm(h_st, f2st[...]) + _mm(h_f, f2sbw[...]) + f2s_b[0:1, :])
    hst_o[...] = h_st
    hf_o[...] = h_f
    p_o[...] = _mm(h_st, mw[...]) + mb[0:1, :]


def mid_body(hst, hf, parts, updt, updbw, upd_b, s2ft, s2fbw, s2f_b,
             f2st, f2sbw, f2s_b, mw, mb, hst_o, hf_o, p_o):
    agg = parts[0:N, :] + parts[NPAD:NPAD + N, :]
    h_st = hst[...]
    h_st = h_st + _relu(_mm(h_st, updt[...]) + _mm(agg, updbw[...]) + upd_b[0:1, :])
    h_f = hf[...]
    h_f = h_f + _relu(_mm(h_f, s2ft[...]) + _mm(h_st, s2fbw[...]) + s2f_b[0:1, :])
    h_st2 = h_st + _relu(_mm(h_st, f2st[...]) + _mm(h_f, f2sbw[...]) + f2s_b[0:1, :])
    hst_o[...] = h_st2
    hf_o[...] = h_f
    p_o[...] = _mm(h_st2, mw[...]) + mb[0:1, :]


def fin_body(hst, hf, parts, updt, updbw, upd_b, s2ft, s2fbw, s2f_b,
             h1t, h1bw, h1_b, h2w, h2b, tot_o, local_o):
    agg = parts[0:N, :] + parts[NPAD:NPAD + N, :]
    h_st = hst[...]
    h_st = h_st + _relu(_mm(h_st, updt[...]) + _mm(agg, updbw[...]) + upd_b[0:1, :])
    h_f = hf[...]
    h_f = h_f + _relu(_mm(h_f, s2ft[...]) + _mm(h_st, s2fbw[...]) + s2f_b[0:1, :])
    z = _relu(_mm(h_st, h1t[...]) + _mm(h_f, h1bw[...]) + h1_b[0:1, :])
    local = _mm(z, h2w[...]) + h2b[0:1, :]
    local_o[...] = local
    tot_o[...] = jnp.sum(local).reshape(1, 1)


def sc_body(p_hbm, q_hbm, src_hbm, dst_hbm, out_hbm,
            isrc, idst, rows, qbuf, acc, gsems, qsems, ssems, isems, dsems):
    c = lax.axis_index("c")
    s = lax.axis_index("s")
    wid = s * NC + c

    def ds8(start, n):
        return pl.ds(pl.multiple_of(start, 8), n)

    zero16 = jnp.zeros((16,), jnp.float32)

    def zrow(i, carry):
        for k in range(H // 16):
            rows[0, i, pl.ds(k * 16, 16)] = zero16
        return carry

    lax.fori_loop(0, C, zrow, 0)
    for j in range(JPT):
        pltpu.sync_copy(rows.at[0], acc.at[pl.ds(s * RPT + j * C, C)])
    plsc.subcore_barrier()

    def issue_idx(g, bi):
        base = ds8(wid * EPW + g * C, C)
        pltpu.async_copy(src_hbm.at[base], isrc.at[bi], isems.at[bi])
        pltpu.async_copy(dst_hbm.at[base], idst.at[bi], dsems.at[bi])

    def wait_idx(g, bi):
        base = ds8(wid * EPW + g * C, C)
        pltpu.make_async_copy(src_hbm.at[base], isrc.at[bi],
                              isems.at[bi]).wait()
        pltpu.make_async_copy(dst_hbm.at[base], idst.at[bi],
                              dsems.at[bi]).wait()

    def issue_gq(g, b, bi):
        pltpu.async_copy(p_hbm.at[isrc.at[bi]], rows.at[b], gsems.at[b])
        pltpu.async_copy(q_hbm.at[ds8(wid * EPW + g * C, C)],
                         qbuf.at[b], qsems.at[b])

    def wait_gq(g, b, bi):
        pltpu.make_async_copy(p_hbm.at[isrc.at[bi]], rows.at[b],
                              gsems.at[b]).wait()
        pltpu.make_async_copy(q_hbm.at[ds8(wid * EPW + g * C, C)],
                              qbuf.at[b], qsems.at[b]).wait()

    def wait_scatter(b, bi):
        pltpu.make_async_copy(rows.at[b], acc.at[idst.at[bi]],
                              ssems.at[b]).wait()

    mask_hi = jnp.full((16,), -65536, jnp.int32)
    sh16 = jnp.full((16,), 16, jnp.int32)

    def compute_scatter(b, bi):
        @plsc.parallel_loop(0, C, unroll=2)
        def erow(e):
            for k in range(H // 32):
                qi = qbuf[b, e, pl.ds(k * 16, 16)]
                lo = jax.lax.bitcast_convert_type(
                    jax.lax.shift_left(qi, sh16), jnp.float32)
                hi = jax.lax.bitcast_convert_type(
                    jax.lax.bitwise_and(qi, mask_hi), jnp.float32)
                sl = pl.ds(k * 32, 16)
                sh = pl.ds(k * 32 + 16, 16)
                rows[b, e, sl] = jnp.maximum(rows[b, e, sl] + lo, 0.0)
                rows[b, e, sh] = jnp.maximum(rows[b, e, sh] + hi, 0.0)
        pltpu.async_copy(rows.at[b], acc.at[idst.at[bi]], ssems.at[b],
                         add=True)

    for x in range(4):
        issue_idx(x, x)
    for x in range(2):
        wait_idx(x, x)
        issue_gq(x, x, x)

    def outer(gg, carry):
        for j in range(IRING):
            g = gg * IRING + j
            b = j % RING

            @pl.when(g + 4 < NCHUNK)
            def _():
                issue_idx(g + 4, (j + 4) % IRING)

            @pl.when(g >= 2)
            def _():
                wait_scatter((j + 2) % RING, (j + 6) % IRING)

            wait_idx(g + 2, (j + 2) % IRING)
            issue_gq(g + 2, (j + 2) % RING, (j + 2) % IRING)

            wait_gq(g, b, j)
            compute_scatter(b, j)
        return carry

    lax.fori_loop(0, NMAIN // IRING, outer, 0)

    for g in range(NMAIN, NCHUNK):
        j = g % IRING
        wait_scatter((j + 2) % RING, (j + 6) % IRING)
        wait_gq(g, j % RING, j)
        compute_scatter(j % RING, j)
    for g in range(NCHUNK - 2, NCHUNK):
        wait_scatter(g % RING, g % IRING)

    plsc.subcore_barrier()
    for j in range(JPT):
        r0 = s * RPT + j * C
        pltpu.sync_copy(acc.at[pl.ds(r0, C)], out_hbm.at[pl.ds(c * NPAD + r0, C)])


def _sc_edge(p, q, src1, dst1):
    return pl.kernel(
        sc_body,
        out_type=jax.ShapeDtypeStruct((2 * NPAD, H), jnp.float32),
        mesh=plsc.VectorSubcoreMesh(core_axis_name="c", subcore_axis_name="s"),
        scratch_types=[
            pltpu.VMEM((IRING, C), jnp.int32),
            pltpu.VMEM((IRING, C), jnp.int32),
            pltpu.VMEM((RING, C, H), jnp.float32),
            pltpu.VMEM((RING, C, H // 2), jnp.int32),
            pltpu.VMEM_SHARED((NPAD, H), jnp.float32),
            pltpu.SemaphoreType.DMA((RING,)),
            pltpu.SemaphoreType.DMA((RING,)),
            pltpu.SemaphoreType.DMA((RING,)),
            pltpu.SemaphoreType.DMA((IRING,)),
            pltpu.SemaphoreType.DMA((IRING,)),
        ],
    )(p, q, src1, dst1)


def _full(shape):
    return pl.BlockSpec(shape, lambda i: tuple(0 for _ in shape))


_TE = 4000


def _q_call(ea, wew, web, bs):
    out = jax.ShapeDtypeStruct((E, H // 2), jnp.int32)
    return pl.pallas_call(
        q_body,
        grid=(E // _TE,),
        in_specs=[
            pl.BlockSpec((_TE, 2), lambda i: (i, 0)),
            _full((2, H)), _full((1, H)),
        ] + [_full((H, H // 2))] * 6,
        out_specs=[pl.BlockSpec((_TE, H // 2), lambda i: (i, 0))] * 3,
        out_shape=[out, out, out],
    )(ea, wew, web, *bs)


def _node_call(body, outs, *args):
    return pl.pallas_call(body, out_shape=list(outs))(*args)


def kernel(st_x, scalar_x, edge_index, edge_attr, params):
    f32 = jnp.float32
    src1 = edge_index[0]
    dst1 = edge_index[1]

    (e1w, e1b), (e2w, e2b) = params["st_enc"]
    (f1w, f1b), (f2w, f2b) = params["f_enc"]
    wew, web = params["edge_enc"]
    blks = params["blocks"]
    (h1w, h1b), (h2w, h2b) = params["head"]

    def row(b):
        return b.reshape(1, -1)

    def split(w):
        return w[:H], w[H:]

    msg_t, msg_b, msg_bias = [], [], []
    f2s_t, f2s_b, f2s_bias = [], [], []
    upd_t, upd_b, upd_bias = [], [], []
    s2f_t, s2f_b, s2f_bias = [], [], []
    for bl in blks:
        t, b_ = split(bl["msg"][0]); msg_t.append(t); msg_b.append(b_)
        msg_bias.append(row(bl["msg"][1]))
        t, b_ = split(bl["f2s"][0]); f2s_t.append(t); f2s_b.append(b_)
        f2s_bias.append(row(bl["f2s"][1]))
        t, b_ = split(bl["upd"][0]); upd_t.append(t); upd_b.append(b_)
        upd_bias.append(row(bl["upd"][1]))
        t, b_ = split(bl["s2f"][0]); s2f_t.append(t); s2f_b.append(b_)
        s2f_bias.append(row(bl["s2f"][1]))
    h1t, h1bw = split(h1w)

    cols_a = [32 * k + j for k in range(H // 32) for j in range(16)]
    cols_b = [c + 16 for c in cols_a]
    bs = []
    for k in range(3):
        bs += [msg_b[k][:, cols_a], msg_b[k][:, cols_b]]
    q1, q2, q3 = _q_call(edge_attr, wew, row(web), bs)

    nh = jax.ShapeDtypeStruct((N, H), f32)
    hst, hf, p1 = _node_call(
        pre_body, (nh, nh, nh),
        st_x, scalar_x, e1w, row(e1b), e2w, row(e2b),
        f1w, row(f1b), f2w, row(f2b),
        f2s_t[0], f2s_b[0], f2s_bias[0], msg_t[0], msg_bias[0],
    )

    parts = _sc_edge(p1, q1, src1, dst1)
    hst, hf, p2 = _node_call(
        mid_body, (nh, nh, nh),
        hst, hf, parts,
        upd_t[0], upd_b[0], upd_bias[0],
        s2f_t[0], s2f_b[0], s2f_bias[0],
        f2s_t[1], f2s_b[1], f2s_bias[1], msg_t[1], msg_bias[1],
    )

    parts = _sc_edge(p2, q2, src1, dst1)
    hst, hf, p3 = _node_call(
        mid_body, (nh, nh, nh),
        hst, hf, parts,
        upd_t[1], upd_b[1], upd_bias[1],
        s2f_t[1], s2f_b[1], s2f_bias[1],
        f2s_t[2], f2s_b[2], f2s_bias[2], msg_t[2], msg_bias[2],
    )

    parts = _sc_edge(p3, q3, src1, dst1)
    tot, local = _node_call(
        fin_body,
        (jax.ShapeDtypeStruct((1, 1), f32), jax.ShapeDtypeStruct((N, 1), f32)),
        hst, hf, parts,
        upd_t[2], upd_b[2], upd_bias[2],
        s2f_t[2], s2f_b[2], s2f_bias[2],
        h1t, h1bw, row(h1b), h2w, row(h2b),
    )
    return tot.reshape(1), local


# speedup vs baseline: 1.0145x; 1.0052x over previous
import jax
import jax.numpy as jnp
from jax import lax
from jax.experimental import pallas as pl
from jax.experimental.pallas import tpu as pltpu
from jax.experimental.pallas import tpu_sc as plsc

H = 128
N = 10000
E = 320000

NC = 2
NS = 16
NW = NC * NS
EPW = E // NW
C = 40
NCHUNK = EPW // C
RING = 4
IRING = 8
NMAIN = (NCHUNK // IRING) * IRING
NPAD = 10240
RPT = NPAD // NS
JPT = RPT // C


def _relu(x):
    return jnp.maximum(x, 0.0)


def _mm(a, b):
    return jax.lax.dot_general(
        a, b, (((1,), (0,)), ((), ())), preferred_element_type=jnp.float32
    )


def q_body(ea, wew, web, b1a, b1b, b2a, b2b, b3a, b3b, q1, q2, q3):
    he = _relu(
        ea[:, 0:1] * wew[0:1, :] + ea[:, 1:2] * wew[1:2, :] + web[0:1, :]
    )

    def packq(ba, bb, qo):
        qa = jax.lax.bitcast_convert_type(
            _mm(he, ba[...]).astype(jnp.bfloat16), jnp.uint16).astype(jnp.int32)
        qb = jax.lax.bitcast_convert_type(
            _mm(he, bb[...]).astype(jnp.bfloat16), jnp.uint16).astype(jnp.int32)
        qo[...] = jax.lax.shift_left(qb, 16) | qa

    packq(b1a, b1b, q1)
    packq(b2a, b2b, q2)
    packq(b3a, b3b, q3)


def pre_body(stx, sx, e1w, e1b, e2w, e2b, f1w, f1b, f2w, f2b,
             f2st, f2sbw, f2s_b, mw, mb, hst_o, hf_o, p_o):
    a = _relu(
        stx[:, 0:1] * e1w[0:1, :]
        + stx[:, 1:2] * e1w[1:2, :]
        + stx[:, 2:3] * e1w[2:3, :]
        + e1b[0:1, :]
    )
    h_st = _relu(_mm(a, e2w[...]) + e2b[0:1, :])
    b0 = _relu(sx[:, 0:1] * f1w[0:1, :] + f1b[0:1, :])
    h_f = _relu(_mm(b0, f2w[...]) + f2b[0:1, :])
    h_st = h_st + _relu(_mm(h_st, f2st[...]) + _mm(h_f, f2sbw[...]) + f2s_b[0:1, :])
    hst_o[...] = h_st
    hf_o[...] = h_f
    p_o[...] = _mm(h_st, mw[...]) + mb[0:1, :]


def mid_body(hst, hf, parts, updt, updbw, upd_b, s2ft, s2fbw, s2f_b,
             f2st, f2sbw, f2s_b, mw, mb, hst_o, hf_o, p_o):
    agg = parts[0:N, :] + parts[NPAD:NPAD + N, :]
    h_st = hst[...]
    h_st = h_st + _relu(_mm(h_st, updt[...]) + _mm(agg, updbw[...]) + upd_b[0:1, :])
    h_f = hf[...]
    h_f = h_f + _relu(_mm(h_f, s2ft[...]) + _mm(h_st, s2fbw[...]) + s2f_b[0:1, :])
    h_st2 = h_st + _relu(_mm(h_st, f2st[...]) + _mm(h_f, f2sbw[...]) + f2s_b[0:1, :])
    hst_o[...] = h_st2
    hf_o[...] = h_f
    p_o[...] = _mm(h_st2, mw[...]) + mb[0:1, :]


def fin_body(hst, hf, parts, updt, updbw, upd_b, s2ft, s2fbw, s2f_b,
             h1t, h1bw, h1_b, h2w, h2b, tot_o, local_o):
    agg = parts[0:N, :] + parts[NPAD:NPAD + N, :]
    h_st = hst[...]
    h_st = h_st + _relu(_mm(h_st, updt[...]) + _mm(agg, updbw[...]) + upd_b[0:1, :])
    h_f = hf[...]
    h_f = h_f + _relu(_mm(h_f, s2ft[...]) + _mm(h_st, s2fbw[...]) + s2f_b[0:1, :])
    z = _relu(_mm(h_st, h1t[...]) + _mm(h_f, h1bw[...]) + h1_b[0:1, :])
    local = _mm(z, h2w[...]) + h2b[0:1, :]
    local_o[...] = local
    tot_o[...] = jnp.sum(local).reshape(1, 1)


def sc_body(p_hbm, q_hbm, src_hbm, dst_hbm, out_hbm,
            isrc, idst, rows, qbuf, acc, gsems, qsems, ssems, isems, dsems):
    c = lax.axis_index("c")
    s = lax.axis_index("s")
    wid = s * NC + c

    def ds8(start, n):
        return pl.ds(pl.multiple_of(start, 8), n)

    zero16 = jnp.zeros((16,), jnp.float32)

    def zrow(i, carry):
        for k in range(H // 16):
            rows[0, i, pl.ds(k * 16, 16)] = zero16
        return carry

    lax.fori_loop(0, C, zrow, 0)
    for j in range(JPT):
        pltpu.sync_copy(rows.at[0], acc.at[pl.ds(s * RPT + j * C, C)])
    plsc.subcore_barrier()

    def issue_idx(g, bi):
        base = ds8(wid * EPW + g * C, C)
        pltpu.async_copy(src_hbm.at[base], isrc.at[bi], isems.at[bi])
        pltpu.async_copy(dst_hbm.at[base], idst.at[bi], dsems.at[bi])

    def wait_idx(g, bi):
        base = ds8(wid * EPW + g * C, C)
        pltpu.make_async_copy(src_hbm.at[base], isrc.at[bi],
                              isems.at[bi]).wait()
        pltpu.make_async_copy(dst_hbm.at[base], idst.at[bi],
                              dsems.at[bi]).wait()

    def issue_gq(g, b, bi):
        pltpu.async_copy(p_hbm.at[isrc.at[bi]], rows.at[b], gsems.at[b])
        pltpu.async_copy(q_hbm.at[ds8(wid * EPW + g * C, C)],
                         qbuf.at[b], qsems.at[b])

    def wait_gq(g, b, bi):
        pltpu.make_async_copy(p_hbm.at[isrc.at[bi]], rows.at[b],
                              gsems.at[b]).wait()
        pltpu.make_async_copy(q_hbm.at[ds8(wid * EPW + g * C, C)],
                              qbuf.at[b], qsems.at[b]).wait()

    def wait_scatter(b, bi):
        pltpu.make_async_copy(rows.at[b], acc.at[idst.at[bi]],
                              ssems.at[b]).wait()

    mask_hi = jnp.full((16,), -65536, jnp.int32)
    sh16 = jnp.full((16,), 16, jnp.int32)

    def compute_scatter(b, bi):
        def erow(e, carry):
            for k in range(H // 32):
                qi = qbuf[b, e, pl.ds(k * 16, 16)]
                lo = jax.lax.bitcast_convert_type(
                    jax.lax.shift_left(qi, sh16), jnp.float32)
                hi = jax.lax.bitcast_convert_type(
                    jax.lax.bitwise_and(qi, mask_hi), jnp.float32)
                sl = pl.ds(k * 32, 16)
                sh = pl.ds(k * 32 + 16, 16)
                rows[b, e, sl] = jnp.maximum(rows[b, e, sl] + lo, 0.0)
                rows[b, e, sh] = jnp.maximum(rows[b, e, sh] + hi, 0.0)
            return carry

        lax.fori_loop(0, C, erow, 0)
        pltpu.async_copy(rows.at[b], acc.at[idst.at[bi]], ssems.at[b],
                         add=True)

    for x in range(4):
        issue_idx(x, x)
    for x in range(2):
        wait_idx(x, x)
        issue_gq(x, x, x)

    def outer(gg, carry):
        for j in range(IRING):
            g = gg * IRING + j
            b = j % RING

            @pl.when(g + 4 < NCHUNK)
            def _():
                issue_idx(g + 4, (j + 4) % IRING)

            @pl.when(g >= 2)
            def _():
                wait_scatter((j + 2) % RING, (j + 6) % IRING)

            wait_idx(g + 2, (j + 2) % IRING)
            issue_gq(g + 2, (j + 2) % RING, (j + 2) % IRING)

            wait_gq(g, b, j)
            compute_scatter(b, j)
        return carry

    lax.fori_loop(0, NMAIN // IRING, outer, 0)

    for g in range(NMAIN, NCHUNK):
        j = g % IRING
        wait_scatter((j + 2) % RING, (j + 6) % IRING)
        wait_gq(g, j % RING, j)
        compute_scatter(j % RING, j)
    for g in range(NCHUNK - 2, NCHUNK):
        wait_scatter(g % RING, g % IRING)

    plsc.subcore_barrier()
    for j in range(JPT):
        r0 = s * RPT + j * C
        pltpu.sync_copy(acc.at[pl.ds(r0, C)], out_hbm.at[pl.ds(c * NPAD + r0, C)])


def _sc_edge(p, q, src1, dst1):
    return pl.kernel(
        sc_body,
        out_type=jax.ShapeDtypeStruct((2 * NPAD, H), jnp.float32),
        mesh=plsc.VectorSubcoreMesh(core_axis_name="c", subcore_axis_name="s"),
        scratch_types=[
            pltpu.VMEM((IRING, C), jnp.int32),
            pltpu.VMEM((IRING, C), jnp.int32),
            pltpu.VMEM((RING, C, H), jnp.float32),
            pltpu.VMEM((RING, C, H // 2), jnp.int32),
            pltpu.VMEM_SHARED((NPAD, H), jnp.float32),
            pltpu.SemaphoreType.DMA((RING,)),
            pltpu.SemaphoreType.DMA((RING,)),
            pltpu.SemaphoreType.DMA((RING,)),
            pltpu.SemaphoreType.DMA((IRING,)),
            pltpu.SemaphoreType.DMA((IRING,)),
        ],
    )(p, q, src1, dst1)


def _full(shape):
    return pl.BlockSpec(shape, lambda i: tuple(0 for _ in shape))


_TE = 4000


def _q_call(ea, wew, web, bs):
    out = jax.ShapeDtypeStruct((E, H // 2), jnp.int32)
    return pl.pallas_call(
        q_body,
        grid=(E // _TE,),
        in_specs=[
            pl.BlockSpec((_TE, 2), lambda i: (i, 0)),
            _full((2, H)), _full((1, H)),
        ] + [_full((H, H // 2))] * 6,
        out_specs=[pl.BlockSpec((_TE, H // 2), lambda i: (i, 0))] * 3,
        out_shape=[out, out, out],
    )(ea, wew, web, *bs)


def _node_call(body, outs, *args):
    return pl.pallas_call(body, out_shape=list(outs))(*args)


def kernel(st_x, scalar_x, edge_index, edge_attr, params):
    f32 = jnp.float32
    src1 = edge_index[0]
    dst1 = edge_index[1]

    (e1w, e1b), (e2w, e2b) = params["st_enc"]
    (f1w, f1b), (f2w, f2b) = params["f_enc"]
    wew, web = params["edge_enc"]
    blks = params["blocks"]
    (h1w, h1b), (h2w, h2b) = params["head"]

    def row(b):
        return b.reshape(1, -1)

    def split(w):
        return w[:H], w[H:]

    cols_a = [32 * k + j for k in range(H // 32) for j in range(16)]
    cols_b = [c + 16 for c in cols_a]
    ilv = []
    for k in range(H // 32):
        for j in range(16):
            ilv += [32 * k + j, 32 * k + 16 + j]

    msg_t, msg_b, msg_bias = [], [], []
    f2s_t, f2s_b, f2s_bias = [], [], []
    upd_t, upd_b, upd_bias = [], [], []
    s2f_t, s2f_b, s2f_bias = [], [], []
    for bl in blks:
        t, b_ = split(bl["msg"][0]); msg_t.append(t); msg_b.append(b_)
        msg_bias.append(row(bl["msg"][1]))
        t, b_ = split(bl["f2s"][0]); f2s_t.append(t); f2s_b.append(b_)
        f2s_bias.append(row(bl["f2s"][1]))
        t, b_ = split(bl["upd"][0]); upd_t.append(t); upd_b.append(b_)
        upd_bias.append(row(bl["upd"][1]))
        t, b_ = split(bl["s2f"][0]); s2f_t.append(t); s2f_b.append(b_)
        s2f_bias.append(row(bl["s2f"][1]))
    h1t, h1bw = split(h1w)

    bs = []
    for k in range(3):
        bs += [msg_b[k][:, cols_a], msg_b[k][:, cols_b]]
    q1, q2, q3 = _q_call(edge_attr, wew, row(web), bs)

    nh = jax.ShapeDtypeStruct((N, H), f32)
    hst, hf, p1 = _node_call(
        pre_body, (nh, nh, nh),
        st_x, scalar_x, e1w, row(e1b), e2w, row(e2b),
        f1w, row(f1b), f2w, row(f2b),
        f2s_t[0], f2s_b[0], f2s_bias[0], msg_t[0], msg_bias[0],
    )

    parts = _sc_edge(p1, q1, src1, dst1)
    hst, hf, p2 = _node_call(
        mid_body, (nh, nh, nh),
        hst, hf, parts,
        upd_t[0], upd_b[0], upd_bias[0],
        s2f_t[0], s2f_b[0], s2f_bias[0],
        f2s_t[1], f2s_b[1], f2s_bias[1], msg_t[1], msg_bias[1],
    )

    parts = _sc_edge(p2, q2, src1, dst1)
    hst, hf, p3 = _node_call(
        mid_body, (nh, nh, nh),
        hst, hf, parts,
        upd_t[1], upd_b[1], upd_bias[1],
        s2f_t[1], s2f_b[1], s2f_bias[1],
        f2s_t[2], f2s_b[2], f2s_bias[2], msg_t[2], msg_bias[2],
    )

    parts = _sc_edge(p3, q3, src1, dst1)
    tot, local = _node_call(
        fin_body,
        (jax.ShapeDtypeStruct((1, 1), f32), jax.ShapeDtypeStruct((N, 1), f32)),
        hst, hf, parts,
        upd_t[2], upd_b[2], upd_bias[2],
        s2f_t[2], s2f_b[2], s2f_bias[2],
        h1t, h1bw, row(h1b), h2w, row(h2b),
    )
    return tot.reshape(1), local


# speedup vs baseline: 1.0184x; 1.0038x over previous
import jax
import jax.numpy as jnp
from jax import lax
from jax.experimental import pallas as pl
from jax.experimental.pallas import tpu as pltpu
from jax.experimental.pallas import tpu_sc as plsc

H = 128
N = 10000
E = 320000

NC = 2
NS = 16
NW = NC * NS
EPW = E // NW
C = 40
NCHUNK = EPW // C
RING = 4
IRING = 8
NMAIN = (NCHUNK // IRING) * IRING
NPAD = 10240
RPT = NPAD // NS
JPT = RPT // C


def _relu(x):
    return jnp.maximum(x, 0.0)


def _mm(a, b):
    return jax.lax.dot_general(
        a, b, (((1,), (0,)), ((), ())), preferred_element_type=jnp.float32
    )


def q_body(ea, wew, web, b1, b2, b3, q1, q2, q3):
    he = _relu(
        ea[:, 0:1] * wew[0:1, :] + ea[:, 1:2] * wew[1:2, :] + web[0:1, :]
    )
    q1[...] = _mm(he, b1[...])
    q2[...] = _mm(he, b2[...])
    q3[...] = _mm(he, b3[...])


def pre_body(stx, sx, e1w, e1b, e2w, e2b, f1w, f1b, f2w, f2b,
             f2st, f2sbw, f2s_b, mw, mb, hst_o, hf_o, p_o):
    a = _relu(
        stx[:, 0:1] * e1w[0:1, :]
        + stx[:, 1:2] * e1w[1:2, :]
        + stx[:, 2:3] * e1w[2:3, :]
        + e1b[0:1, :]
    )
    h_st = _relu(_mm(a, e2w[...]) + e2b[0:1, :])
    b0 = _relu(sx[:, 0:1] * f1w[0:1, :] + f1b[0:1, :])
    h_f = _relu(_mm(b0, f2w[...]) + f2b[0:1, :])
    h_st = h_st + _relu(_mm(h_st, f2st[...]) + _mm(h_f, f2sbw[...]) + f2s_b[0:1, :])
    hst_o[...] = h_st
    hf_o[...] = h_f
    p_o[...] = _mm(h_st, mw[...]) + mb[0:1, :]


def mid_body(hst, hf, parts, updt, updbw, upd_b, s2ft, s2fbw, s2f_b,
             f2st, f2sbw, f2s_b, mw, mb, hst_o, hf_o, p_o):
    agg = parts[0:N, :] + parts[NPAD:NPAD + N, :]
    h_st = hst[...]
    h_st = h_st + _relu(_mm(h_st, updt[...]) + _mm(agg, updbw[...]) + upd_b[0:1, :])
    h_f = hf[...]
    h_f = h_f + _relu(_mm(h_f, s2ft[...]) + _mm(h_st, s2fbw[...]) + s2f_b[0:1, :])
    h_st2 = h_st + _relu(_mm(h_st, f2st[...]) + _mm(h_f, f2sbw[...]) + f2s_b[0:1, :])
    hst_o[...] = h_st2
    hf_o[...] = h_f
    p_o[...] = _mm(h_st2, mw[...]) + mb[0:1, :]


def fin_body(hst, hf, parts, updt, updbw, upd_b, s2ft, s2fbw, s2f_b,
             h1t, h1bw, h1_b, h2w, h2b, tot_o, local_o):
    agg = parts[0:N, :] + parts[NPAD:NPAD + N, :]
    h_st = hst[...]
    h_st = h_st + _relu(_mm(h_st, updt[...]) + _mm(agg, updbw[...]) + upd_b[0:1, :])
    h_f = hf[...]
    h_f = h_f + _relu(_mm(h_f, s2ft[...]) + _mm(h_st, s2fbw[...]) + s2f_b[0:1, :])
    z = _relu(_mm(h_st, h1t[...]) + _mm(h_f, h1bw[...]) + h1_b[0:1, :])
    local = _mm(z, h2w[...]) + h2b[0:1, :]
    local_o[...] = local
    tot_o[...] = jnp.sum(local).reshape(1, 1)


def sc_body(p_hbm, q_hbm, src_hbm, dst_hbm, out_hbm,
            isrc, idst, rows, qbuf, acc, gsems, qsems, ssems, isems, dsems):
    c = lax.axis_index("c")
    s = lax.axis_index("s")
    wid = s * NC + c

    def ds8(start, n):
        return pl.ds(pl.multiple_of(start, 8), n)

    zero16 = jnp.zeros((16,), jnp.float32)

    def zrow(i, carry):
        for k in range(H // 16):
            rows[0, i, pl.ds(k * 16, 16)] = zero16
        return carry

    lax.fori_loop(0, C, zrow, 0)
    for j in range(JPT):
        pltpu.sync_copy(rows.at[0], acc.at[pl.ds(s * RPT + j * C, C)])
    plsc.subcore_barrier()

    def issue_idx(g, bi):
        base = ds8(wid * EPW + g * C, C)
        pltpu.async_copy(src_hbm.at[base], isrc.at[bi], isems.at[bi])
        pltpu.async_copy(dst_hbm.at[base], idst.at[bi], dsems.at[bi])

    def wait_idx(g, bi):
        base = ds8(wid * EPW + g * C, C)
        pltpu.make_async_copy(src_hbm.at[base], isrc.at[bi],
                              isems.at[bi]).wait()
        pltpu.make_async_copy(dst_hbm.at[base], idst.at[bi],
                              dsems.at[bi]).wait()

    def issue_gq(g, b, bi):
        pltpu.async_copy(p_hbm.at[isrc.at[bi]], rows.at[b], gsems.at[b])
        pltpu.async_copy(q_hbm.at[ds8(wid * EPW + g * C, C)],
                         qbuf.at[b], qsems.at[b])

    def wait_gq(g, b, bi):
        pltpu.make_async_copy(p_hbm.at[isrc.at[bi]], rows.at[b],
                              gsems.at[b]).wait()
        pltpu.make_async_copy(q_hbm.at[ds8(wid * EPW + g * C, C)],
                              qbuf.at[b], qsems.at[b]).wait()

    def wait_scatter(b, bi):
        pltpu.make_async_copy(rows.at[b], acc.at[idst.at[bi]],
                              ssems.at[b]).wait()

    def compute_scatter(b, bi):
        def erow(e, carry):
            for k in range(H // 16):
                sl = pl.ds(k * 16, 16)
                rows[b, e, sl] = jnp.maximum(
                    rows[b, e, sl] + qbuf[b, e, sl], 0.0)
            return carry

        lax.fori_loop(0, C, erow, 0)
        pltpu.async_copy(rows.at[b], acc.at[idst.at[bi]], ssems.at[b],
                         add=True)

    for x in range(4):
        issue_idx(x, x)
    for x in range(2):
        wait_idx(x, x)
        issue_gq(x, x, x)

    def outer(gg, carry):
        for j in range(IRING):
            g = gg * IRING + j
            b = j % RING

            @pl.when(g + 4 < NCHUNK)
            def _():
                issue_idx(g + 4, (j + 4) % IRING)

            @pl.when(g >= 2)
            def _():
                wait_scatter((j + 2) % RING, (j + 6) % IRING)

            wait_idx(g + 2, (j + 2) % IRING)
            issue_gq(g + 2, (j + 2) % RING, (j + 2) % IRING)

            wait_gq(g, b, j)
            compute_scatter(b, j)
        return carry

    lax.fori_loop(0, NMAIN // IRING, outer, 0)

    for g in range(NMAIN, NCHUNK):
        j = g % IRING
        wait_scatter((j + 2) % RING, (j + 6) % IRING)
        wait_gq(g, j % RING, j)
        compute_scatter(j % RING, j)
    for g in range(NCHUNK - 2, NCHUNK):
        wait_scatter(g % RING, g % IRING)

    plsc.subcore_barrier()
    for j in range(JPT):
        r0 = s * RPT + j * C
        pltpu.sync_copy(acc.at[pl.ds(r0, C)], out_hbm.at[pl.ds(c * NPAD + r0, C)])


def _sc_edge(p, q, src1, dst1):
    return pl.kernel(
        sc_body,
        out_type=jax.ShapeDtypeStruct((2 * NPAD, H), jnp.float32),
        mesh=plsc.VectorSubcoreMesh(core_axis_name="c", subcore_axis_name="s"),
        scratch_types=[
            pltpu.VMEM((IRING, C), jnp.int32),
            pltpu.VMEM((IRING, C), jnp.int32),
            pltpu.VMEM((RING, C, H), jnp.float32),
            pltpu.VMEM((RING, C, H), jnp.float32),
            pltpu.VMEM_SHARED((NPAD, H), jnp.float32),
            pltpu.SemaphoreType.DMA((RING,)),
            pltpu.SemaphoreType.DMA((RING,)),
            pltpu.SemaphoreType.DMA((RING,)),
            pltpu.SemaphoreType.DMA((IRING,)),
            pltpu.SemaphoreType.DMA((IRING,)),
        ],
    )(p, q, src1, dst1)


def _full(shape):
    return pl.BlockSpec(shape, lambda i: tuple(0 for _ in shape))


_TE = 4000


def _q_call(ea, wew, web, bs):
    out = jax.ShapeDtypeStruct((E, H), jnp.float32)
    return pl.pallas_call(
        q_body,
        grid=(E // _TE,),
        in_specs=[
            pl.BlockSpec((_TE, 2), lambda i: (i, 0)),
            _full((2, H)), _full((1, H)),
        ] + [_full((H, H))] * 3,
        out_specs=[pl.BlockSpec((_TE, H), lambda i: (i, 0))] * 3,
        out_shape=[out, out, out],
    )(ea, wew, web, *bs)


def _node_call(body, outs, *args):
    return pl.pallas_call(body, out_shape=list(outs))(*args)


def kernel(st_x, scalar_x, edge_index, edge_attr, params):
    f32 = jnp.float32
    src1 = edge_index[0]
    dst1 = edge_index[1]

    (e1w, e1b), (e2w, e2b) = params["st_enc"]
    (f1w, f1b), (f2w, f2b) = params["f_enc"]
    wew, web = params["edge_enc"]
    blks = params["blocks"]
    (h1w, h1b), (h2w, h2b) = params["head"]

    def row(b):
        return b.reshape(1, -1)

    def split(w):
        return w[:H], w[H:]

    msg_t, msg_b, msg_bias = [], [], []
    f2s_t, f2s_b, f2s_bias = [], [], []
    upd_t, upd_b, upd_bias = [], [], []
    s2f_t, s2f_b, s2f_bias = [], [], []
    for bl in blks:
        t, b_ = split(bl["msg"][0]); msg_t.append(t); msg_b.append(b_)
        msg_bias.append(row(bl["msg"][1]))
        t, b_ = split(bl["f2s"][0]); f2s_t.append(t); f2s_b.append(b_)
        f2s_bias.append(row(bl["f2s"][1]))
        t, b_ = split(bl["upd"][0]); upd_t.append(t); upd_b.append(b_)
        upd_bias.append(row(bl["upd"][1]))
        t, b_ = split(bl["s2f"][0]); s2f_t.append(t); s2f_b.append(b_)
        s2f_bias.append(row(bl["s2f"][1]))
    h1t, h1bw = split(h1w)

    q1, q2, q3 = _q_call(edge_attr, wew, row(web), msg_b)

    nh = jax.ShapeDtypeStruct((N, H), f32)
    hst, hf, p1 = _node_call(
        pre_body, (nh, nh, nh),
        st_x, scalar_x, e1w, row(e1b), e2w, row(e2b),
        f1w, row(f1b), f2w, row(f2b),
        f2s_t[0], f2s_b[0], f2s_bias[0], msg_t[0], msg_bias[0],
    )

    parts = _sc_edge(p1, q1, src1, dst1)
    hst, hf, p2 = _node_call(
        mid_body, (nh, nh, nh),
        hst, hf, parts,
        upd_t[0], upd_b[0], upd_bias[0],
        s2f_t[0], s2f_b[0], s2f_bias[0],
        f2s_t[1], f2s_b[1], f2s_bias[1], msg_t[1], msg_bias[1],
    )

    parts = _sc_edge(p2, q2, src1, dst1)
    hst, hf, p3 = _node_call(
        mid_body, (nh, nh, nh),
        hst, hf, parts,
        upd_t[1], upd_b[1], upd_bias[1],
        s2f_t[1], s2f_b[1], s2f_bias[1],
        f2s_t[2], f2s_b[2], f2s_bias[2], msg_t[2], msg_bias[2],
    )

    parts = _sc_edge(p3, q3, src1, dst1)
    tot, local = _node_call(
        fin_body,
        (jax.ShapeDtypeStruct((1, 1), f32), jax.ShapeDtypeStruct((N, 1), f32)),
        hst, hf, parts,
        upd_t[2], upd_b[2], upd_bias[2],
        s2f_t[2], s2f_b[2], s2f_bias[2],
        h1t, h1bw, row(h1b), h2w, row(h2b),
    )
    return tot.reshape(1), local
